# baseline (speedup 1.0000x reference)
import functools

import jax
import jax.numpy as jnp
from jax import lax
from jax.experimental import pallas as pl
from jax.experimental.pallas import tpu as pltpu
from jax.experimental.pallas import tpu_sc as plsc

B = 4096
L = 200
DIM = 32
N = B * L
NC = 2
NS = 16
NW = NC * NS
NBB = B // 128
NITEMS = L * NBB
PER_W = NITEMS // NW
NBUF = 4
NOUTER = PER_W // NBUF
IDX_PER_W = PER_W * 128


def _sc_embed(idx1, table):
    mesh = plsc.VectorSubcoreMesh(core_axis_name="c", subcore_axis_name="s")

    @functools.partial(
        pl.kernel,
        out_type=jax.ShapeDtypeStruct((L, DIM // 8, B // 128, 8, 128), jnp.float32),
        mesh=mesh,
        scratch_types=[
            pltpu.VMEM((IDX_PER_W,), jnp.int32),
            pltpu.VMEM((NBUF, 128, DIM), jnp.float32),
            pltpu.VMEM((NBUF, DIM // 8, 8, 128), jnp.float32),
            pltpu.SemaphoreType.DMA((NBUF,)),
            pltpu.SemaphoreType.DMA((NBUF,)),
        ],
        compiler_params=pltpu.CompilerParams(
            use_tc_tiling_on_sc=False, needs_layout_passes=False
        ),
    )
    def k(idx_hbm, table_hbm, out_hbm, idx_v, rows_v, slab_v, sem_g, sem_o):
        wid = lax.axis_index("s") * NC + lax.axis_index("c")
        t0 = wid * PER_W

        pltpu.sync_copy(idx_hbm.at[pl.ds(t0 * 128, IDX_PER_W)], idx_v)

        def gather(i, b):
            return pltpu.async_copy(
                table_hbm.at[idx_v.at[pl.ds(i * 128, 128)]],
                rows_v.at[b],
                sem_g.at[b],
            )

        def write(i, b):
            t = t0 + i
            l = t // NBB
            bb = t % NBB
            return pltpu.async_copy(
                slab_v.at[b],
                out_hbm.at[l, :, bb],
                sem_o.at[b],
            )

        row_ids = [jnp.arange(16, dtype=jnp.int32) + (16 * g) for g in range(8)]

        def transpose(b):
            for db in range(DIM // 8):
                for dq in range(8):
                    d = 8 * db + dq
                    col = jnp.full((16,), d, dtype=jnp.int32)
                    for g in range(8):
                        v = plsc.load_gather(rows_v.at[b], [row_ids[g], col])
                        slab_v[b, db, dq, pl.ds(16 * g, 16)] = v

        for b in range(NBUF):
            gather(b, b)

        def outer(g, _):
            for b in range(NBUF):
                i = g * NBUF + b
                pltpu.make_async_copy(
                    table_hbm.at[idx_v.at[pl.ds(0, 128)]],
                    rows_v.at[b],
                    sem_g.at[b],
                ).wait()

                @pl.when(g > 0)
                def _():
                    pltpu.make_async_copy(
                        slab_v.at[b],
                        out_hbm.at[0, :, 0],
                        sem_o.at[b],
                    ).wait()

                transpose(b)
                write(i, b)

                @pl.when(g < NOUTER - 1)
                def _():
                    gather(i + NBUF, b)

            return ()

        lax.fori_loop(0, NOUTER, outer, ())

        for b in range(NBUF):
            pltpu.make_async_copy(
                slab_v.at[b],
                out_hbm.at[0, :, 0],
                sem_o.at[b],
            ).wait()

    return k(idx1, table)


def kernel(inputs, table):
    idx1 = inputs.T.reshape(N)
    out5 = _sc_embed(idx1, table)
    return out5.transpose(2, 4, 0, 1, 3).reshape(B, L, DIM)


# speedup vs baseline: 1.3462x; 1.3462x over previous
import functools

import jax
import jax.numpy as jnp
from jax import lax
from jax.experimental import pallas as pl
from jax.experimental.pallas import tpu as pltpu
from jax.experimental.pallas import tpu_sc as plsc

B = 4096
L = 200
DIM = 32
N = B * L
NC = 2
NS = 16
NW = NC * NS
NBUF = 4
NOUTER = L // NBUF


def _sc_embed(inputs, table):
    mesh = plsc.VectorSubcoreMesh(core_axis_name="c", subcore_axis_name="s")

    @functools.partial(
        pl.kernel,
        out_type=jax.ShapeDtypeStruct((L, DIM // 8, B // 128, 8, 128), jnp.float32),
        mesh=mesh,
        scratch_types=[
            pltpu.VMEM((128, L), jnp.int32),
            pltpu.VMEM((L, 128), jnp.int32),
            pltpu.VMEM((NBUF, 128, DIM), jnp.float32),
            pltpu.VMEM((NBUF, DIM, 128), jnp.float32),
            pltpu.SemaphoreType.DMA((NBUF,)),
            pltpu.SemaphoreType.DMA((NBUF,)),
        ],
        compiler_params=pltpu.CompilerParams(
            use_tc_tiling_on_sc=False, needs_layout_passes=False
        ),
    )
    def k(idx_hbm, table_hbm, out_hbm, idx_v, idxt_v, rows_v, slab_v, sem_g, sem_o):
        wid = lax.axis_index("s") * NC + lax.axis_index("c")

        pltpu.sync_copy(idx_hbm.at[pl.ds(wid * 128, 128)], idx_v)

        iota = lax.iota(jnp.int32, 16)

        def idx_tr(l, _):
            for g in range(8):
                rows = iota + (16 * g)
                v = plsc.load_gather(idx_v, [rows, jnp.full((16,), 0, jnp.int32) + l])
                idxt_v[l, pl.ds(16 * g, 16)] = v
            return ()

        lax.fori_loop(0, L, idx_tr, ())

        def gather(i, b):
            return pltpu.async_copy(
                table_hbm.at[idxt_v.at[i]],
                rows_v.at[b],
                sem_g.at[b],
            )

        def write(i, b):
            cps = []
            for db in range(DIM // 8):
                cps.append(pltpu.async_copy(
                    slab_v.at[b].at[pl.ds(8 * db, 8)],
                    out_hbm.at[i, db, wid],
                    sem_o.at[b],
                ))
            return cps

        def wait_write(b):
            for db in range(DIM // 8):
                pltpu.make_async_copy(
                    slab_v.at[b].at[pl.ds(8 * db, 8)],
                    out_hbm.at[0, db, 0],
                    sem_o.at[b],
                ).wait()

        def transpose(b):
            rows2 = rows_v.at[b]
            slab2 = slab_v.at[b]
            for kk in range(DIM):
                dvec = (iota + kk) & (DIM - 1)
                for g in range(8):
                    bvec = iota + (16 * g)
                    v = plsc.load_gather(rows2, [bvec, dvec])
                    plsc.store_scatter(slab2, [dvec, bvec], v)

        for b in range(NBUF):
            gather(b, b)

        def outer(g, _):
            for b in range(NBUF):
                i = g * NBUF + b
                pltpu.make_async_copy(
                    table_hbm.at[idxt_v.at[0]],
                    rows_v.at[b],
                    sem_g.at[b],
                ).wait()

                @pl.when(g > 0)
                def _():
                    wait_write(b)

                transpose(b)
                write(i, b)

                @pl.when(g < NOUTER - 1)
                def _():
                    gather(i + NBUF, b)

            return ()

        lax.fori_loop(0, NOUTER, outer, ())

        for b in range(NBUF):
            wait_write(b)

    return k(inputs, table)


def kernel(inputs, table):
    out5 = _sc_embed(inputs, table)
    return out5.transpose(2, 4, 0, 1, 3).reshape(B, L, DIM)


# speedup vs baseline: 1.3517x; 1.0041x over previous
import functools

import jax
import jax.numpy as jnp
from jax import lax
from jax.experimental import pallas as pl
from jax.experimental.pallas import tpu as pltpu
from jax.experimental.pallas import tpu_sc as plsc

B = 4096
L = 200
DIM = 32
N = B * L
NC = 2
NS = 16
NW = NC * NS
NBUF = 4
NOUTER = L // NBUF


def _sc_embed(idx4, table):
    mesh = plsc.VectorSubcoreMesh(core_axis_name="c", subcore_axis_name="s")

    @functools.partial(
        pl.kernel,
        out_type=jax.ShapeDtypeStruct((L, DIM // 8, B // 128, 8, 128), jnp.float32),
        mesh=mesh,
        scratch_types=[
            pltpu.VMEM((L // 8, 8, 128), jnp.int32),
            pltpu.VMEM((NBUF, 128, DIM), jnp.float32),
            pltpu.VMEM((NBUF, DIM, 128), jnp.float32),
            pltpu.SemaphoreType.DMA,
            pltpu.SemaphoreType.DMA((NBUF,)),
            pltpu.SemaphoreType.DMA((NBUF,)),
        ],
        compiler_params=pltpu.CompilerParams(
            use_tc_tiling_on_sc=False, needs_layout_passes=False
        ),
    )
    def k(idx_hbm, table_hbm, out_hbm, idx_v, rows_v, slab_v, sem_i, sem_g, sem_o):
        wid = lax.axis_index("s") * NC + lax.axis_index("c")

        pltpu.async_copy(idx_hbm.at[:, wid], idx_v, sem_i).wait()

        iota = lax.iota(jnp.int32, 16)

        def gather(i, b):
            return pltpu.async_copy(
                table_hbm.at[idx_v.at[i // 8, i % 8]],
                rows_v.at[b],
                sem_g.at[b],
            )

        def write(i, b):
            for db in range(DIM // 8):
                pltpu.async_copy(
                    slab_v.at[b].at[pl.ds(8 * db, 8)],
                    out_hbm.at[i, db, wid],
                    sem_o.at[b],
                )

        def wait_write(b):
            for db in range(DIM // 8):
                pltpu.make_async_copy(
                    slab_v.at[b].at[pl.ds(8 * db, 8)],
                    out_hbm.at[0, db, 0],
                    sem_o.at[b],
                ).wait()

        def transpose(b):
            rows2 = rows_v.at[b]
            slab2 = slab_v.at[b]
            for kk in range(DIM):
                dvec = (iota + kk) & (DIM - 1)
                for g in range(8):
                    bvec = iota + (16 * g)
                    v = plsc.load_gather(rows2, [bvec, dvec])
                    plsc.store_scatter(slab2, [dvec, bvec], v)

        for b in range(NBUF):
            gather(b, b)

        def outer(g, _):
            for b in range(NBUF):
                i = g * NBUF + b
                pltpu.make_async_copy(
                    table_hbm.at[idx_v.at[0, 0]],
                    rows_v.at[b],
                    sem_g.at[b],
                ).wait()

                @pl.when(g > 0)
                def _():
                    wait_write(b)

                transpose(b)
                write(i, b)

                @pl.when(g < NOUTER - 1)
                def _():
                    gather(i + NBUF, b)

            return ()

        lax.fori_loop(0, NOUTER, outer, ())

        for b in range(NBUF):
            wait_write(b)

    return k(idx4, table)


def kernel(inputs, table):
    idx4 = inputs.T.reshape(L // 8, 8, B // 128, 128).transpose(0, 2, 1, 3)
    out5 = _sc_embed(idx4, table)
    return out5.transpose(2, 4, 0, 1, 3).reshape(B, L, DIM)


# speedup vs baseline: 2.0971x; 1.5514x over previous
import functools

import jax
import jax.numpy as jnp
from jax import lax
from jax.experimental import pallas as pl
from jax.experimental.pallas import tpu as pltpu
from jax.experimental.pallas import tpu_sc as plsc

B = 4096
L = 200
DIM = 32
N = B * L
NC = 2
NS = 16
NW = NC * NS
NBUF = 4
NOUTER = L // NBUF
NTAIL = L - NOUTER * NBUF


def _sc_embed(idx4, table128):
    mesh = plsc.VectorSubcoreMesh(core_axis_name="c", subcore_axis_name="s")

    @functools.partial(
        pl.kernel,
        out_type=jax.ShapeDtypeStruct((L, DIM // 8, B // 128, 8, 128), jnp.float32),
        mesh=mesh,
        scratch_types=[
            pltpu.VMEM((L // 8, 8, 128), jnp.int32),
            pltpu.VMEM((NBUF, 128, DIM), jnp.float32),
            pltpu.VMEM((NBUF, DIM, 128), jnp.float32),
            pltpu.SemaphoreType.DMA,
            pltpu.SemaphoreType.DMA((NBUF,)),
            pltpu.SemaphoreType.DMA((NBUF,)),
        ],
        compiler_params=pltpu.CompilerParams(
            use_tc_tiling_on_sc=False,
            needs_layout_passes=False,
            disable_bounds_checks=True,
        ),
    )
    def k(idx_hbm, table_hbm, out_hbm, idx_v, rows_v, slab_v,
          sem_i, sem_g, sem_o):
        wid = lax.axis_index("s") * NC + lax.axis_index("c")

        pltpu.async_copy(idx_hbm.at[:, wid], idx_v, sem_i).wait()

        iota = lax.iota(jnp.int32, 16)

        def gather(i, b):
            return pltpu.async_copy(
                table_hbm.at[idx_v.at[i // 8, i % 8]],
                rows_v.at[b],
                sem_g.at[b],
            )

        def write(i, b):
            for db in range(DIM // 8):
                pltpu.async_copy(
                    slab_v.at[b].at[pl.ds(8 * db, 8)],
                    out_hbm.at[i, db, wid],
                    sem_o.at[b],
                )

        def wait_write(b):
            for db in range(DIM // 8):
                pltpu.make_async_copy(
                    slab_v.at[b].at[pl.ds(8 * db, 8)],
                    out_hbm.at[0, db, 0],
                    sem_o.at[b],
                ).wait()

        def extract(i, b):
            rows2 = rows_v.at[b]
            slab2 = slab_v.at[b]

            def kkbody(kk, dvec):
                work = []
                for g in range(8):
                    bvec = iota + (16 * g)
                    work.append((bvec, plsc.load_gather(rows2, [bvec, dvec])))
                for bvec, v in work:
                    plsc.store_scatter(slab2, [dvec, bvec], v)
                return (dvec + 1) & (DIM - 1)

            lax.fori_loop(0, DIM, kkbody, iota & (DIM - 1))

        for b in range(NBUF):
            gather(b, b)

        def outer(g, _):
            for b in range(NBUF):
                i = g * NBUF + b
                pltpu.make_async_copy(
                    table_hbm.at[idx_v.at[0, 0]],
                    rows_v.at[b],
                    sem_g.at[b],
                ).wait()

                @pl.when(g > 0)
                def _():
                    wait_write(b)

                extract(i, b)
                write(i, b)

                @pl.when(i + NBUF < L)
                def _():
                    gather(i + NBUF, b)

            return ()

        lax.fori_loop(0, NOUTER, outer, ())

        for t in range(NTAIL):
            i = NOUTER * NBUF + t
            b = i % NBUF
            pltpu.make_async_copy(
                table_hbm.at[idx_v.at[0, 0]],
                rows_v.at[b],
                sem_g.at[b],
            ).wait()
            wait_write(b)
            extract(i, b)
            write(i, b)

        for b in range(NBUF):
            wait_write(b)

    return k(idx4, table128)


def kernel(inputs, table):
    idx4 = inputs.T.reshape(L // 8, 8, B // 128, 128).transpose(0, 2, 1, 3)
    out5 = _sc_embed(idx4, table)
    return out5.transpose(2, 4, 0, 1, 3).reshape(B, L, DIM)


# speedup vs baseline: 3.3622x; 1.6033x over previous
import functools

import jax
import jax.numpy as jnp
from jax import lax
from jax.experimental import pallas as pl
from jax.experimental.pallas import tpu as pltpu
from jax.experimental.pallas import tpu_sc as plsc

B = 4096
L = 200
DIM = 32
N = B * L
NC = 2
NS = 16
NW = NC * NS
NBUF = 4
NOUTER = L // NBUF
NTAIL = L - NOUTER * NBUF


CHR = 512
NCHUNK = 999936 // CHR


def _sc_reformat(table_t):
    mesh = plsc.VectorSubcoreMesh(core_axis_name="c", subcore_axis_name="s")

    @functools.partial(
        pl.kernel,
        out_type=jax.ShapeDtypeStruct((250000, 128), jnp.float32),
        mesh=mesh,
        scratch_types=[
            pltpu.VMEM((2, DIM, CHR), jnp.float32),
            pltpu.VMEM((2, CHR // 4, 128), jnp.float32),
            pltpu.VMEM((DIM, 64), jnp.float32),
            pltpu.VMEM((16, 128), jnp.float32),
            pltpu.SemaphoreType.DMA((2,)),
            pltpu.SemaphoreType.DMA((2,)),
        ],
        compiler_params=pltpu.CompilerParams(
            needs_layout_passes=False,
            disable_bounds_checks=True,
        ),
    )
    def ka(tab_hbm, out_hbm, inb, outb, in64, out64, sem_r, sem_w):
        wid = lax.axis_index("s") * NC + lax.axis_index("c")
        iota = lax.iota(jnp.int32, 16)

        def r0_of(c):
            return pl.multiple_of(c * CHR, 128)

        def read(c, b):
            return pltpu.async_copy(
                tab_hbm.at[:, pl.ds(r0_of(c), CHR)], inb.at[b], sem_r.at[b])

        def write(c, b):
            return pltpu.async_copy(
                outb.at[b],
                out_hbm.at[pl.ds(pl.multiple_of(r0_of(c) // 4, 8), CHR // 4), :],
                sem_w.at[b])

        def transpose(b):
            in2 = inb.at[b]
            out2 = outb.at[b]

            def tbody(t, _):
                rg = lax.shift_right_logical(t, 3)
                kk2 = t & 7
                rrvec = iota + lax.shift_left(rg, 4)
                srow = lax.shift_right_logical(rrvec, 2)
                scol = lax.shift_left(rrvec & 3, 5)
                dvec = (iota + lax.shift_left(kk2, 2)) & (DIM - 1)
                vals = []
                for _q in range(4):
                    vals.append((dvec, plsc.load_gather(in2, [dvec, rrvec])))
                    dvec = (dvec + 1) & (DIM - 1)
                for dv, v in vals:
                    plsc.store_scatter(out2, [srow, scol + dv], v)
                return ()

            lax.fori_loop(0, (CHR // 16) * 8, tbody, ())

        nmine = (NCHUNK - wid + NW - 1) // NW

        def body(j, _):
            b = j % 2
            c = wid + NW * j
            read(c, b).wait()

            @pl.when(j >= 2)
            def _():
                pltpu.make_async_copy(
                    outb.at[b],
                    out_hbm.at[pl.ds(0, CHR // 4), :],
                    sem_w.at[b],
                ).wait()

            transpose(b)
            write(c, b)
            return ()

        lax.fori_loop(0, nmine, body, ())

        for b in range(2):
            @pl.when(nmine >= b + 1)
            def _():
                pltpu.make_async_copy(
                    outb.at[b],
                    out_hbm.at[pl.ds(0, CHR // 4), :],
                    sem_w.at[b],
                ).wait()

        @pl.when(wid == 0)
        def _():
            pltpu.async_copy(
                tab_hbm.at[:, pl.ds(999936, 64)], in64, sem_r.at[0]).wait()

            def t64(t, _):
                rg = lax.shift_right_logical(t, 3)
                kk2 = t & 7
                rrvec = iota + lax.shift_left(rg, 4)
                srow = lax.shift_right_logical(rrvec, 2)
                scol = lax.shift_left(rrvec & 3, 5)
                dvec = (iota + lax.shift_left(kk2, 2)) & (DIM - 1)
                vals = []
                for _q in range(4):
                    vals.append((dvec, plsc.load_gather(in64, [dvec, rrvec])))
                    dvec = (dvec + 1) & (DIM - 1)
                for dv, v in vals:
                    plsc.store_scatter(out64, [srow, scol + dv], v)
                return ()

            lax.fori_loop(0, (64 // 16) * 8, t64, ())
            pltpu.async_copy(
                out64, out_hbm.at[pl.ds(249984, 16), :], sem_w.at[0]).wait()


    return ka(table_t)


def _sc_embed(idx4, table128):
    mesh = plsc.VectorSubcoreMesh(core_axis_name="c", subcore_axis_name="s")

    @functools.partial(
        pl.kernel,
        out_type=jax.ShapeDtypeStruct((L, DIM // 8, B // 128, 8, 128), jnp.float32),
        mesh=mesh,
        scratch_types=[
            pltpu.VMEM((L // 8, 8, 128), jnp.int32),
            pltpu.VMEM((NBUF, 128, DIM), jnp.float32),
            pltpu.VMEM((NBUF, DIM, 128), jnp.float32),
            pltpu.SemaphoreType.DMA,
            pltpu.SemaphoreType.DMA((NBUF,)),
            pltpu.SemaphoreType.DMA((NBUF,)),
        ],
        compiler_params=pltpu.CompilerParams(
            use_tc_tiling_on_sc=False,
            needs_layout_passes=False,
            disable_bounds_checks=True,
        ),
    )
    def k(idx_hbm, table_hbm, out_hbm, idx_v, rows_v, slab_v,
          sem_i, sem_g, sem_o):
        wid = lax.axis_index("s") * NC + lax.axis_index("c")

        pltpu.async_copy(idx_hbm.at[:, wid], idx_v, sem_i).wait()

        iota = lax.iota(jnp.int32, 16)

        def gather(i, b):
            return pltpu.async_copy(
                table_hbm.at[idx_v.at[i // 8, i % 8]],
                rows_v.at[b],
                sem_g.at[b],
            )

        def write(i, b):
            for db in range(DIM // 8):
                pltpu.async_copy(
                    slab_v.at[b].at[pl.ds(8 * db, 8)],
                    out_hbm.at[i, db, wid],
                    sem_o.at[b],
                )

        def wait_write(b):
            for db in range(DIM // 8):
                pltpu.make_async_copy(
                    slab_v.at[b].at[pl.ds(8 * db, 8)],
                    out_hbm.at[0, db, 0],
                    sem_o.at[b],
                ).wait()

        def extract(i, b):
            rows2 = rows_v.at[b]
            slab2 = slab_v.at[b]

            def kkbody(kk, dvec):
                work = []
                for g in range(8):
                    bvec = iota + (16 * g)
                    work.append((bvec, plsc.load_gather(rows2, [bvec, dvec])))
                for bvec, v in work:
                    plsc.store_scatter(slab2, [dvec, bvec], v)
                return (dvec + 1) & (DIM - 1)

            lax.fori_loop(0, DIM, kkbody, iota & (DIM - 1))

        for b in range(NBUF):
            gather(b, b)

        def outer(g, _):
            for b in range(NBUF):
                i = g * NBUF + b
                pltpu.make_async_copy(
                    table_hbm.at[idx_v.at[0, 0]],
                    rows_v.at[b],
                    sem_g.at[b],
                ).wait()

                @pl.when(g > 0)
                def _():
                    wait_write(b)

                extract(i, b)
                write(i, b)

                @pl.when(i + NBUF < L)
                def _():
                    gather(i + NBUF, b)

            return ()

        lax.fori_loop(0, NOUTER, outer, ())

        for t in range(NTAIL):
            i = NOUTER * NBUF + t
            b = i % NBUF
            pltpu.make_async_copy(
                table_hbm.at[idx_v.at[0, 0]],
                rows_v.at[b],
                sem_g.at[b],
            ).wait()
            wait_write(b)
            extract(i, b)
            write(i, b)

        for b in range(NBUF):
            wait_write(b)

    return k(idx4, table128)


def kernel(inputs, table):
    idx4 = inputs.T.reshape(L // 8, 8, B // 128, 128).transpose(0, 2, 1, 3)
    tfmt = _sc_reformat(table.T)
    out5 = _sc_embed(idx4, tfmt.reshape(1000000, DIM))
    return out5.transpose(2, 4, 0, 1, 3).reshape(B, L, DIM)


# speedup vs baseline: 3.9357x; 1.1706x over previous
import functools

import jax
import jax.numpy as jnp
from jax import lax
from jax.experimental import pallas as pl
from jax.experimental.pallas import tpu as pltpu
from jax.experimental.pallas import tpu_sc as plsc

B = 4096
L = 200
DIM = 32
N = B * L
NC = 2
NS = 16
NW = NC * NS
NBUF = 4
NOUTER = L // NBUF
NTAIL = L - NOUTER * NBUF


CHR = 768
NCHUNK = 999936 // CHR


def _sc_reformat(table_t):
    mesh = plsc.VectorSubcoreMesh(core_axis_name="c", subcore_axis_name="s")

    @functools.partial(
        pl.kernel,
        out_type=jax.ShapeDtypeStruct((250000, 128), jnp.float32),
        mesh=mesh,
        scratch_types=[
            pltpu.VMEM((2, DIM, CHR), jnp.float32),
            pltpu.VMEM((2, CHR // 4, 128), jnp.float32),
            pltpu.VMEM((DIM, 64), jnp.float32),
            pltpu.VMEM((16, 128), jnp.float32),
            pltpu.SemaphoreType.DMA((2,)),
            pltpu.SemaphoreType.DMA((2,)),
        ],
        compiler_params=pltpu.CompilerParams(
            needs_layout_passes=False,
            disable_bounds_checks=True,
        ),
    )
    def ka(tab_hbm, out_hbm, inb, outb, in64, out64, sem_r, sem_w):
        wid = lax.axis_index("s") * NC + lax.axis_index("c")
        iota = lax.iota(jnp.int32, 16)

        def r0_of(c):
            return pl.multiple_of(c * CHR, 128)

        def read(c, b):
            return pltpu.async_copy(
                tab_hbm.at[:, pl.ds(r0_of(c), CHR)], inb.at[b], sem_r.at[b])

        def write(c, b):
            return pltpu.async_copy(
                outb.at[b],
                out_hbm.at[pl.ds(pl.multiple_of(r0_of(c) // 4, 8), CHR // 4), :],
                sem_w.at[b])

        def transpose(b):
            in2 = inb.at[b]
            out2 = outb.at[b]

            def tbody(t, _):
                rg = lax.shift_right_logical(t, 2)
                kk2 = t & 3
                rrvec = iota + lax.shift_left(rg, 4)
                srow = lax.shift_right_logical(rrvec, 2)
                scol = lax.shift_left(rrvec & 3, 5)
                dvec = (iota + lax.shift_left(kk2, 3)) & (DIM - 1)
                vals = []
                for _q in range(8):
                    vals.append((dvec, plsc.load_gather(in2, [dvec, rrvec])))
                    dvec = (dvec + 1) & (DIM - 1)
                for dv, v in vals:
                    plsc.store_scatter(out2, [srow, scol + dv], v)
                return ()

            lax.fori_loop(0, (CHR // 16) * 4, tbody, ())

        nmine = (NCHUNK - wid + NW - 1) // NW

        def body(j, _):
            b = j % 2
            c = wid + NW * j
            read(c, b).wait()

            @pl.when(j >= 2)
            def _():
                pltpu.make_async_copy(
                    outb.at[b],
                    out_hbm.at[pl.ds(0, CHR // 4), :],
                    sem_w.at[b],
                ).wait()

            transpose(b)
            write(c, b)
            return ()

        lax.fori_loop(0, nmine, body, ())

        for b in range(2):
            @pl.when(nmine >= b + 1)
            def _():
                pltpu.make_async_copy(
                    outb.at[b],
                    out_hbm.at[pl.ds(0, CHR // 4), :],
                    sem_w.at[b],
                ).wait()

        @pl.when(wid == 0)
        def _():
            pltpu.async_copy(
                tab_hbm.at[:, pl.ds(999936, 64)], in64, sem_r.at[0]).wait()

            def t64(t, _):
                rg = lax.shift_right_logical(t, 3)
                kk2 = t & 7
                rrvec = iota + lax.shift_left(rg, 4)
                srow = lax.shift_right_logical(rrvec, 2)
                scol = lax.shift_left(rrvec & 3, 5)
                dvec = (iota + lax.shift_left(kk2, 2)) & (DIM - 1)
                vals = []
                for _q in range(4):
                    vals.append((dvec, plsc.load_gather(in64, [dvec, rrvec])))
                    dvec = (dvec + 1) & (DIM - 1)
                for dv, v in vals:
                    plsc.store_scatter(out64, [srow, scol + dv], v)
                return ()

            lax.fori_loop(0, (64 // 16) * 8, t64, ())
            pltpu.async_copy(
                out64, out_hbm.at[pl.ds(249984, 16), :], sem_w.at[0]).wait()


    return ka(table_t)


def _sc_embed(idx4, table128):
    mesh = plsc.VectorSubcoreMesh(core_axis_name="c", subcore_axis_name="s")

    @functools.partial(
        pl.kernel,
        out_type=jax.ShapeDtypeStruct((L, DIM // 8, B // 128, 8, 128), jnp.float32),
        mesh=mesh,
        scratch_types=[
            pltpu.VMEM((L // 8, 8, 128), jnp.int32),
            pltpu.VMEM((NBUF, 128, DIM), jnp.float32),
            pltpu.VMEM((NBUF, DIM, 128), jnp.float32),
            pltpu.SemaphoreType.DMA,
            pltpu.SemaphoreType.DMA((NBUF,)),
            pltpu.SemaphoreType.DMA((NBUF,)),
        ],
        compiler_params=pltpu.CompilerParams(
            use_tc_tiling_on_sc=False,
            needs_layout_passes=False,
            disable_bounds_checks=True,
        ),
    )
    def k(idx_hbm, table_hbm, out_hbm, idx_v, rows_v, slab_v,
          sem_i, sem_g, sem_o):
        wid = lax.axis_index("s") * NC + lax.axis_index("c")

        pltpu.async_copy(idx_hbm.at[:, wid], idx_v, sem_i).wait()

        iota = lax.iota(jnp.int32, 16)

        def gather(i, b):
            return pltpu.async_copy(
                table_hbm.at[idx_v.at[i // 8, i % 8]],
                rows_v.at[b],
                sem_g.at[b],
            )

        def write(i, b):
            for db in range(DIM // 8):
                pltpu.async_copy(
                    slab_v.at[b].at[pl.ds(8 * db, 8)],
                    out_hbm.at[i, db, wid],
                    sem_o.at[b],
                )

        def wait_write(b):
            for db in range(DIM // 8):
                pltpu.make_async_copy(
                    slab_v.at[b].at[pl.ds(8 * db, 8)],
                    out_hbm.at[0, db, 0],
                    sem_o.at[b],
                ).wait()

        def extract(i, b):
            rows2 = rows_v.at[b]
            slab2 = slab_v.at[b]

            def kkbody(kk, dvec):
                work = []
                for g in range(8):
                    bvec = iota + (16 * g)
                    work.append((bvec, plsc.load_gather(rows2, [bvec, dvec])))
                for bvec, v in work:
                    plsc.store_scatter(slab2, [dvec, bvec], v)
                return (dvec + 1) & (DIM - 1)

            lax.fori_loop(0, DIM, kkbody, iota & (DIM - 1))

        for b in range(NBUF):
            gather(b, b)

        def outer(g, _):
            for b in range(NBUF):
                i = g * NBUF + b
                pltpu.make_async_copy(
                    table_hbm.at[idx_v.at[0, 0]],
                    rows_v.at[b],
                    sem_g.at[b],
                ).wait()

                @pl.when(g > 0)
                def _():
                    wait_write(b)

                extract(i, b)
                write(i, b)

                @pl.when(i + NBUF < L)
                def _():
                    gather(i + NBUF, b)

            return ()

        lax.fori_loop(0, NOUTER, outer, ())

        for t in range(NTAIL):
            i = NOUTER * NBUF + t
            b = i % NBUF
            pltpu.make_async_copy(
                table_hbm.at[idx_v.at[0, 0]],
                rows_v.at[b],
                sem_g.at[b],
            ).wait()
            wait_write(b)
            extract(i, b)
            write(i, b)

        for b in range(NBUF):
            wait_write(b)

    return k(idx4, table128)


def kernel(inputs, table):
    idx4 = inputs.T.reshape(L // 8, 8, B // 128, 128).transpose(0, 2, 1, 3)
    tfmt = _sc_reformat(table.T)
    out5 = _sc_embed(idx4, tfmt.reshape(1000000, DIM))
    return out5.transpose(2, 4, 0, 1, 3).reshape(B, L, DIM)


# speedup vs baseline: 4.0487x; 1.0287x over previous
import functools

import jax
import jax.numpy as jnp
from jax import lax
from jax.experimental import pallas as pl
from jax.experimental.pallas import tpu as pltpu
from jax.experimental.pallas import tpu_sc as plsc

B = 4096
L = 200
DIM = 32
N = B * L
NC = 2
NS = 16
NW = NC * NS
NBUF = 4
NOUTER = L // NBUF
NTAIL = L - NOUTER * NBUF


CHR = 768
NCHUNK = 999936 // CHR


def _sc_reformat(table_t):
    mesh = plsc.VectorSubcoreMesh(core_axis_name="c", subcore_axis_name="s")

    @functools.partial(
        pl.kernel,
        out_type=jax.ShapeDtypeStruct((250000, 128), jnp.float32),
        mesh=mesh,
        scratch_types=[
            pltpu.VMEM((2, DIM, CHR), jnp.float32),
            pltpu.VMEM((2, CHR // 4, 128), jnp.float32),
            pltpu.VMEM((DIM, 64), jnp.float32),
            pltpu.VMEM((16, 128), jnp.float32),
            pltpu.SemaphoreType.DMA((2,)),
            pltpu.SemaphoreType.DMA((2,)),
        ],
        compiler_params=pltpu.CompilerParams(
            needs_layout_passes=False,
            disable_bounds_checks=True,
        ),
    )
    def ka(tab_hbm, out_hbm, inb, outb, in64, out64, sem_r, sem_w):
        wid = lax.axis_index("s") * NC + lax.axis_index("c")
        iota = lax.iota(jnp.int32, 16)

        def r0_of(c):
            return pl.multiple_of(c * CHR, 128)

        def read(c, b):
            return pltpu.async_copy(
                tab_hbm.at[:, pl.ds(r0_of(c), CHR)], inb.at[b], sem_r.at[b])

        def write(c, b):
            return pltpu.async_copy(
                outb.at[b],
                out_hbm.at[pl.ds(pl.multiple_of(r0_of(c) // 4, 8), CHR // 4), :],
                sem_w.at[b])

        def transpose(b):
            in2 = inb.at[b]
            out2 = outb.at[b]

            def tbody(t, _):
                rg = lax.shift_right_logical(t, 1)
                kk2 = t & 1
                rrvec = iota + lax.shift_left(rg, 4)
                srow = lax.shift_right_logical(rrvec, 2)
                scol = lax.shift_left(rrvec & 3, 5)
                dvec = (iota + lax.shift_left(kk2, 4)) & (DIM - 1)
                vals = []
                for _q in range(16):
                    vals.append((dvec, plsc.load_gather(in2, [dvec, rrvec])))
                    dvec = (dvec + 1) & (DIM - 1)
                for dv, v in vals:
                    plsc.store_scatter(out2, [srow, scol + dv], v)
                return ()

            lax.fori_loop(0, (CHR // 16) * 2, tbody, ())

        nmine = (NCHUNK - wid + NW - 1) // NW

        def body(j, _):
            b = j % 2
            c = wid + NW * j
            read(c, b).wait()

            @pl.when(j >= 2)
            def _():
                pltpu.make_async_copy(
                    outb.at[b],
                    out_hbm.at[pl.ds(0, CHR // 4), :],
                    sem_w.at[b],
                ).wait()

            transpose(b)
            write(c, b)
            return ()

        lax.fori_loop(0, nmine, body, ())

        for b in range(2):
            @pl.when(nmine >= b + 1)
            def _():
                pltpu.make_async_copy(
                    outb.at[b],
                    out_hbm.at[pl.ds(0, CHR // 4), :],
                    sem_w.at[b],
                ).wait()

        @pl.when(wid == 0)
        def _():
            pltpu.async_copy(
                tab_hbm.at[:, pl.ds(999936, 64)], in64, sem_r.at[0]).wait()

            def t64(t, _):
                rg = lax.shift_right_logical(t, 3)
                kk2 = t & 7
                rrvec = iota + lax.shift_left(rg, 4)
                srow = lax.shift_right_logical(rrvec, 2)
                scol = lax.shift_left(rrvec & 3, 5)
                dvec = (iota + lax.shift_left(kk2, 2)) & (DIM - 1)
                vals = []
                for _q in range(4):
                    vals.append((dvec, plsc.load_gather(in64, [dvec, rrvec])))
                    dvec = (dvec + 1) & (DIM - 1)
                for dv, v in vals:
                    plsc.store_scatter(out64, [srow, scol + dv], v)
                return ()

            lax.fori_loop(0, (64 // 16) * 8, t64, ())
            pltpu.async_copy(
                out64, out_hbm.at[pl.ds(249984, 16), :], sem_w.at[0]).wait()


    return ka(table_t)


def _sc_embed(idx4, table128):
    mesh = plsc.VectorSubcoreMesh(core_axis_name="c", subcore_axis_name="s")

    @functools.partial(
        pl.kernel,
        out_type=jax.ShapeDtypeStruct((L, DIM // 8, B // 128, 8, 128), jnp.float32),
        mesh=mesh,
        scratch_types=[
            pltpu.VMEM((L // 8, 8, 128), jnp.int32),
            pltpu.VMEM((NBUF, 128, DIM), jnp.float32),
            pltpu.VMEM((NBUF, DIM, 128), jnp.float32),
            pltpu.SemaphoreType.DMA,
            pltpu.SemaphoreType.DMA((NBUF,)),
            pltpu.SemaphoreType.DMA((NBUF,)),
        ],
        compiler_params=pltpu.CompilerParams(
            use_tc_tiling_on_sc=False,
            needs_layout_passes=False,
            disable_bounds_checks=True,
        ),
    )
    def k(idx_hbm, table_hbm, out_hbm, idx_v, rows_v, slab_v,
          sem_i, sem_g, sem_o):
        wid = lax.axis_index("s") * NC + lax.axis_index("c")

        pltpu.async_copy(idx_hbm.at[:, wid], idx_v, sem_i).wait()

        iota = lax.iota(jnp.int32, 16)

        def gather(i, b):
            return pltpu.async_copy(
                table_hbm.at[idx_v.at[i // 8, i % 8]],
                rows_v.at[b],
                sem_g.at[b],
            )

        def write(i, b):
            for db in range(DIM // 8):
                pltpu.async_copy(
                    slab_v.at[b].at[pl.ds(8 * db, 8)],
                    out_hbm.at[i, db, wid],
                    sem_o.at[b],
                )

        def wait_write(b):
            for db in range(DIM // 8):
                pltpu.make_async_copy(
                    slab_v.at[b].at[pl.ds(8 * db, 8)],
                    out_hbm.at[0, db, 0],
                    sem_o.at[b],
                ).wait()

        def extract(i, b):
            rows2 = rows_v.at[b]
            slab2 = slab_v.at[b]

            def kkbody(kk, dvec):
                work = []
                for g in range(8):
                    bvec = iota + (16 * g)
                    work.append((bvec, plsc.load_gather(rows2, [bvec, dvec])))
                for bvec, v in work:
                    plsc.store_scatter(slab2, [dvec, bvec], v)
                return (dvec + 1) & (DIM - 1)

            lax.fori_loop(0, DIM, kkbody, iota & (DIM - 1))

        for b in range(NBUF):
            gather(b, b)

        def outer(g, _):
            for b in range(NBUF):
                i = g * NBUF + b
                pltpu.make_async_copy(
                    table_hbm.at[idx_v.at[0, 0]],
                    rows_v.at[b],
                    sem_g.at[b],
                ).wait()

                @pl.when(g > 0)
                def _():
                    wait_write(b)

                extract(i, b)
                write(i, b)

                @pl.when(i + NBUF < L)
                def _():
                    gather(i + NBUF, b)

            return ()

        lax.fori_loop(0, NOUTER, outer, ())

        for t in range(NTAIL):
            i = NOUTER * NBUF + t
            b = i % NBUF
            pltpu.make_async_copy(
                table_hbm.at[idx_v.at[0, 0]],
                rows_v.at[b],
                sem_g.at[b],
            ).wait()
            wait_write(b)
            extract(i, b)
            write(i, b)

        for b in range(NBUF):
            wait_write(b)

    return k(idx4, table128)


def kernel(inputs, table):
    idx4 = inputs.T.reshape(L // 8, 8, B // 128, 128).transpose(0, 2, 1, 3)
    tfmt = _sc_reformat(table.T)
    out5 = _sc_embed(idx4, tfmt.reshape(1000000, DIM))
    return out5.transpose(2, 4, 0, 1, 3).reshape(B, L, DIM)
